# baseline (speedup 1.0000x reference)
import functools

import jax
import jax.numpy as jnp
from jax import lax
from jax.experimental import pallas as pl
from jax.experimental.pallas import tpu as pltpu
from jax.experimental.pallas import tpu_sc as plsc

N_CODES = 8192
DIM = 256
TOK_BLK = 512
CODE_BLK = 512
BETA = 0.25


def _vq_argmin_body(z_ref, sz2_ref, emb_ref, se2_ref, idx_ref, dsum_ref):
    zneg = z_ref[...] * -2.0
    sz2 = sz2_ref[...]
    nchunk = N_CODES // CODE_BLK
    group = 2
    rmin = jnp.full((TOK_BLK, CODE_BLK), jnp.inf, dtype=jnp.float32)
    rchunk = jnp.zeros((TOK_BLK, CODE_BLK), dtype=jnp.int32)
    for g in range(nchunk // group):
        ds = []
        for k in range(group):
            c = g * group + k
            emb_c = emb_ref[pl.ds(c * CODE_BLK, CODE_BLK), :]
            se2_c = se2_ref[:, pl.ds(c * CODE_BLK, CODE_BLK)]
            mm = lax.dot_general(zneg, emb_c, (((1,), (1,)), ((), ())),
                                 preferred_element_type=jnp.float32)
            ds.append((sz2 + se2_c) + mm)
        m01 = ds[1] < ds[0]
        dg = jnp.minimum(ds[0], ds[1])
        cg = jnp.where(m01, g * group + 1, g * group)
        better = dg < rmin
        rchunk = jnp.where(better, cg, rchunk)
        rmin = jnp.minimum(rmin, dg)
    m = jnp.min(rmin, axis=1)
    jj = lax.broadcasted_iota(jnp.int32, (TOK_BLK, CODE_BLK), 1)
    n = rchunk * CODE_BLK + jj
    run_idx = jnp.min(jnp.where(rmin == m[:, None], n, N_CODES), axis=1)
    idx_ref[0, 0, :] = run_idx
    dsum_ref[0, 0, :] = jnp.full((TOK_BLK,), jnp.sum(m), dtype=jnp.float32)


def _argmin_distances(z_flat, sz2, emb_weight, se2_row):
    nb = z_flat.shape[0] // TOK_BLK
    return pl.pallas_call(
        _vq_argmin_body,
        grid=(nb,),
        in_specs=[
            pl.BlockSpec((TOK_BLK, DIM), lambda i: (i, 0)),
            pl.BlockSpec((TOK_BLK, 1), lambda i: (i, 0)),
            pl.BlockSpec((N_CODES, DIM), lambda i: (0, 0)),
            pl.BlockSpec((1, N_CODES), lambda i: (0, 0)),
        ],
        out_specs=[
            pl.BlockSpec((1, 1, TOK_BLK), lambda i: (i, 0, 0)),
            pl.BlockSpec((1, 1, TOK_BLK), lambda i: (i, 0, 0)),
        ],
        out_shape=[
            jax.ShapeDtypeStruct((nb, 1, TOK_BLK), jnp.int32),
            jax.ShapeDtypeStruct((nb, 1, TOK_BLK), jnp.float32),
        ],
    )(z_flat, sz2, emb_weight, se2_row)


def _gather_rows_sc(emb_weight, idx2d, n_tokens):
    info = plsc.get_sparse_core_info()
    nw = info.num_cores * info.num_subcores
    bpw = n_tokens // nw
    nchunk = bpw // 128
    mesh = plsc.VectorSubcoreMesh(core_axis_name="c", subcore_axis_name="s")

    @functools.partial(
        pl.kernel, mesh=mesh,
        out_type=jax.ShapeDtypeStruct((n_tokens, DIM), jnp.float32),
        scratch_types=[
            pltpu.VMEM((nchunk, 128), jnp.int32),
            pltpu.VMEM((bpw, DIM), jnp.float32),
            pltpu.SemaphoreType.DMA,
        ],
    )
    def k(table_hbm, idx_hbm, out_hbm, idx_v, rows_v, sem):
        wid = lax.axis_index("s") * info.num_cores + lax.axis_index("c")
        pltpu.sync_copy(idx_hbm.at[pl.ds(wid * nchunk, nchunk)], idx_v)
        copies = [
            pltpu.async_copy(table_hbm.at[idx_v.at[j]],
                             rows_v.at[pl.ds(j * 128, 128)], sem)
            for j in range(nchunk)
        ]
        for cp in copies:
            cp.wait()
        pltpu.sync_copy(rows_v, out_hbm.at[pl.ds(wid * bpw, bpw)])

    return k(emb_weight, idx2d)


def kernel(z, emb_weight):
    zt = jnp.transpose(z, (0, 1, 3, 4, 2))
    z_flat = zt.reshape(-1, DIM)
    n_tok = z_flat.shape[0]
    sz2 = jnp.sum(z_flat ** 2, axis=1, keepdims=True)
    se2_row = jnp.sum(emb_weight ** 2, axis=1).reshape(1, N_CODES)

    idx3, dsum3 = _argmin_distances(z_flat, sz2, emb_weight, se2_row)
    indices = idx3.reshape(n_tok)
    loss = (1.0 + BETA) * jnp.sum(dsum3[:, 0, 0]) / float(z.size)

    z_q_flat = _gather_rows_sc(emb_weight, indices.reshape(-1, 128), n_tok)
    z_q = jnp.transpose(z_q_flat.reshape(zt.shape), (0, 1, 4, 2, 3))
    return z_q, loss, indices


# speedup vs baseline: 1.1865x; 1.1865x over previous
import functools

import jax
import jax.numpy as jnp
from jax import lax
from jax.experimental import pallas as pl
from jax.experimental.pallas import tpu as pltpu
from jax.experimental.pallas import tpu_sc as plsc

N_CODES = 8192
DIM = 256
TOK_BLK = 512
CODE_BLK = 256
BETA = 0.25


def _vq_argmin_body(z_ref, sz2_ref, emb_ref, se2_ref, idx_ref, dsum_ref):
    zneg = z_ref[...] * -2.0
    sz2 = sz2_ref[...]
    nchunk = N_CODES // CODE_BLK
    rmin = jnp.full((TOK_BLK, CODE_BLK), jnp.inf, dtype=jnp.float32)
    rchunk = jnp.zeros((TOK_BLK, CODE_BLK), dtype=jnp.int32)
    for c in range(nchunk):
        emb_c = emb_ref[pl.ds(c * CODE_BLK, CODE_BLK), :]
        se2_c = se2_ref[:, pl.ds(c * CODE_BLK, CODE_BLK)]
        mm = lax.dot_general(zneg, emb_c, (((1,), (1,)), ((), ())),
                             preferred_element_type=jnp.float32)
        d = (sz2 + se2_c) + mm
        better = d < rmin
        rchunk = jnp.where(better, c, rchunk)
        rmin = jnp.minimum(rmin, d)
    m = jnp.min(rmin, axis=1)
    jj = lax.broadcasted_iota(jnp.int32, (TOK_BLK, CODE_BLK), 1)
    n = rchunk * CODE_BLK + jj
    run_idx = jnp.min(jnp.where(rmin == m[:, None], n, N_CODES), axis=1)
    idx_ref[0, 0, :] = run_idx
    dsum_ref[0, 0, :] = jnp.full((TOK_BLK,), jnp.sum(m), dtype=jnp.float32)


def _argmin_distances(z_flat, sz2, emb_weight, se2_row):
    nb = z_flat.shape[0] // TOK_BLK
    return pl.pallas_call(
        _vq_argmin_body,
        grid=(nb,),
        in_specs=[
            pl.BlockSpec((TOK_BLK, DIM), lambda i: (i, 0)),
            pl.BlockSpec((TOK_BLK, 1), lambda i: (i, 0)),
            pl.BlockSpec((N_CODES, DIM), lambda i: (0, 0)),
            pl.BlockSpec((1, N_CODES), lambda i: (0, 0)),
        ],
        out_specs=[
            pl.BlockSpec((1, 1, TOK_BLK), lambda i: (i, 0, 0)),
            pl.BlockSpec((1, 1, TOK_BLK), lambda i: (i, 0, 0)),
        ],
        out_shape=[
            jax.ShapeDtypeStruct((nb, 1, TOK_BLK), jnp.int32),
            jax.ShapeDtypeStruct((nb, 1, TOK_BLK), jnp.float32),
        ],
    )(z_flat, sz2, emb_weight, se2_row)


def _gather_rows_sc(emb_weight, idx2d, n_tokens):
    info = plsc.get_sparse_core_info()
    nw = info.num_cores * info.num_subcores
    bpw = n_tokens // nw
    nchunk = bpw // 128
    mesh = plsc.VectorSubcoreMesh(core_axis_name="c", subcore_axis_name="s")

    @functools.partial(
        pl.kernel, mesh=mesh,
        out_type=jax.ShapeDtypeStruct((n_tokens, DIM), jnp.float32),
        scratch_types=[
            pltpu.VMEM((nchunk, 128), jnp.int32),
            pltpu.VMEM((bpw, DIM), jnp.float32),
            pltpu.SemaphoreType.DMA,
        ],
    )
    def k(table_hbm, idx_hbm, out_hbm, idx_v, rows_v, sem):
        wid = lax.axis_index("s") * info.num_cores + lax.axis_index("c")
        pltpu.sync_copy(idx_hbm.at[pl.ds(wid * nchunk, nchunk)], idx_v)
        copies = [
            pltpu.async_copy(table_hbm.at[idx_v.at[j]],
                             rows_v.at[pl.ds(j * 128, 128)], sem)
            for j in range(nchunk)
        ]
        for cp in copies:
            cp.wait()
        pltpu.sync_copy(rows_v, out_hbm.at[pl.ds(wid * bpw, bpw)])

    return k(emb_weight, idx2d)


def kernel(z, emb_weight):
    zt = jnp.transpose(z, (0, 1, 3, 4, 2))
    z_flat = zt.reshape(-1, DIM)
    n_tok = z_flat.shape[0]
    sz2 = jnp.sum(z_flat ** 2, axis=1, keepdims=True)
    se2_row = jnp.sum(emb_weight ** 2, axis=1).reshape(1, N_CODES)

    idx3, dsum3 = _argmin_distances(z_flat, sz2, emb_weight, se2_row)
    indices = idx3.reshape(n_tok)
    loss = (1.0 + BETA) * jnp.sum(dsum3[:, 0, 0]) / float(z.size)

    z_q_flat = _gather_rows_sc(emb_weight, indices.reshape(-1, 128), n_tok)
    z_q = jnp.transpose(z_q_flat.reshape(zt.shape), (0, 1, 4, 2, 3))
    return z_q, loss, indices
